# baseline (speedup 1.0000x reference)
import jax
import jax.numpy as jnp
from jax import lax
from jax.experimental import pallas as pl
from jax.experimental.pallas import tpu as pltpu

B, N, M = 8, 2048, 2048


def _chamfer_body(a_ref, b_ref, out_ref):
    bi = pl.program_id(0)
    a = a_ref[0]
    b = b_ref[0]
    ab = lax.dot_general(a, b, (((1,), (1,)), ((), ())),
                         preferred_element_type=jnp.float32)
    a2 = jnp.sum(a * a, axis=1, keepdims=True)
    b2r = jnp.sum(b * b, axis=1)[None, :]
    d = (a2 + b2r) - 2.0 * ab
    s1 = jnp.sum(jnp.maximum(jnp.min(d, axis=1), 0.0))
    s2 = jnp.sum(jnp.maximum(jnp.min(d, axis=0), 0.0))
    inc = jnp.reshape(s1 + s2, (1, 1))

    @pl.when(bi == 0)
    def _init():
        out_ref[...] = inc

    @pl.when(bi > 0)
    def _acc():
        out_ref[...] += inc


def kernel(array1, array2):
    out = pl.pallas_call(
        _chamfer_body,
        grid=(B,),
        in_specs=[
            pl.BlockSpec((1, N, 3), lambda i: (i, 0, 0)),
            pl.BlockSpec((1, M, 3), lambda i: (i, 0, 0)),
        ],
        out_specs=pl.BlockSpec((1, 1), lambda i: (0, 0)),
        out_shape=jax.ShapeDtypeStruct((1, 1), jnp.float32),
    )(array1, array2)
    return out[0, 0] * (1.0 / (B * N))


# speedup vs baseline: 1.6537x; 1.6537x over previous
import jax
import jax.numpy as jnp
from jax import lax
from jax.experimental import pallas as pl
from jax.experimental.pallas import tpu as pltpu

B, N, M = 8, 2048, 2048
K = 5


def _chamfer_body(l_ref, r_ref, out_ref):
    bi = pl.program_id(0)
    l = l_ref[0]
    r = r_ref[0]
    d = lax.dot_general(l, r, (((0,), (0,)), ((), ())),
                        preferred_element_type=jnp.float32)
    s1 = jnp.sum(jnp.maximum(jnp.min(d, axis=1), 0.0))
    s2 = jnp.sum(jnp.maximum(jnp.min(d, axis=0), 0.0))
    inc = jnp.reshape(s1 + s2, (1, 1))

    @pl.when(bi == 0)
    def _init():
        out_ref[...] = inc

    @pl.when(bi > 0)
    def _acc():
        out_ref[...] += inc


def kernel(array1, array2):
    a_t = jnp.transpose(array1, (0, 2, 1))
    b_t = jnp.transpose(array2, (0, 2, 1))
    a2 = jnp.sum(a_t * a_t, axis=1, keepdims=True)
    b2 = jnp.sum(b_t * b_t, axis=1, keepdims=True)
    ones_a = jnp.ones_like(a2)
    l_aug = jnp.concatenate([a_t, a2, ones_a], axis=1)
    r_aug = jnp.concatenate([-2.0 * b_t, ones_a, b2], axis=1)
    out = pl.pallas_call(
        _chamfer_body,
        grid=(B,),
        in_specs=[
            pl.BlockSpec((1, K, N), lambda i: (i, 0, 0)),
            pl.BlockSpec((1, K, M), lambda i: (i, 0, 0)),
        ],
        out_specs=pl.BlockSpec((1, 1), lambda i: (0, 0)),
        out_shape=jax.ShapeDtypeStruct((1, 1), jnp.float32),
    )(l_aug, r_aug)
    return out[0, 0] * (1.0 / (B * N))
